# baseline (speedup 1.0000x reference)
import functools

import jax
import jax.numpy as jnp
from jax import lax
from jax.experimental import pallas as pl
from jax.experimental.pallas import tpu as pltpu
from jax.experimental.pallas import tpu_sc as plsc

N_NODES = 10000
N_EDGES = 320000
ATOM_DIM = 128
EDGE_DIM = 16
HIDDEN = 128

NC = 2
NS = 16
NW = NC * NS

NH = 2
EH = N_EDGES // NH
EPW = EH // NW

GCB = 40
GCHUNKS = EPW // GCB
KB = 5
GOUTER = GCHUNKS // KB

SCB = 128
SFULL = EPW // SCB
STAIL = EPW - SFULL * SCB
SOUTER = SFULL

AGG_ROWS = 10240


def _sc_gather(x, ej3):
    mesh = plsc.VectorSubcoreMesh(core_axis_name="c", subcore_axis_name="s")

    @functools.partial(
        pl.kernel, mesh=mesh,
        out_type=jax.ShapeDtypeStruct((EH, ATOM_DIM), jnp.float32),
        scratch_types=[
            pltpu.VMEM((GCHUNKS, GCB), jnp.int32),
            pltpu.VMEM((KB, GCB, ATOM_DIM), jnp.float32),
            pltpu.SemaphoreType.DMA((KB,)),
            pltpu.SemaphoreType.DMA((KB,)),
        ],
    )
    def k(x_hbm, ej_hbm, out_hbm, idx_v, rows_v, gsem, wsem):
        wid = lax.axis_index("s") * NC + lax.axis_index("c")
        base = wid * EPW
        pltpu.sync_copy(ej_hbm.at[wid], idx_v)

        def outer(t, _):
            for b in range(KB):
                j = t * KB + b

                @pl.when(t > 0)
                def _wait_prev_write():
                    pltpu.make_async_copy(
                        rows_v.at[b], out_hbm.at[pl.ds(base + j * GCB, GCB)],
                        wsem.at[b]).wait()

                pltpu.make_async_copy(
                    x_hbm.at[idx_v.at[j]], rows_v.at[b], gsem.at[b]).start()
            for b in range(KB):
                j = t * KB + b
                pltpu.make_async_copy(
                    x_hbm.at[idx_v.at[j]], rows_v.at[b], gsem.at[b]).wait()
                pltpu.make_async_copy(
                    rows_v.at[b], out_hbm.at[pl.ds(base + j * GCB, GCB)],
                    wsem.at[b]).start()
            return 0

        lax.fori_loop(0, GOUTER, outer, 0)
        for b in range(KB):
            pltpu.make_async_copy(
                rows_v.at[b], out_hbm.at[pl.ds(base + b * GCB, GCB)],
                wsem.at[b]).wait()

    return k(x, ej3)


def _sc_scatter_add(m, ei_main, ei_tail, zeros_nh):
    mesh = plsc.VectorSubcoreMesh(core_axis_name="c", subcore_axis_name="s")
    rows_per_sub = AGG_ROWS // NS

    @functools.partial(
        pl.kernel, mesh=mesh,
        out_type=jax.ShapeDtypeStruct((NC, AGG_ROWS, HIDDEN), jnp.float32),
        scratch_types=[
            pltpu.VMEM((SFULL, SCB), jnp.int32),
            pltpu.VMEM((STAIL,), jnp.int32),
            pltpu.VMEM((2, SCB, HIDDEN), jnp.float32),
            pltpu.VMEM((STAIL, HIDDEN), jnp.float32),
            pltpu.VMEM_SHARED((AGG_ROWS, HIDDEN), jnp.float32),
            pltpu.SemaphoreType.DMA((2,)),
            pltpu.SemaphoreType.DMA((2,)),
        ],
    )
    def k(m_hbm, ei_hbm, eit_hbm, z_hbm, out_hbm,
          idx_v, idxt_v, rows_v, rowst_v, agg_sh, msem, ssem):
        cid = lax.axis_index("c")
        sid = lax.axis_index("s")
        wid = sid * NC + cid
        base = wid * EPW
        pltpu.sync_copy(ei_hbm.at[wid], idx_v)
        pltpu.sync_copy(eit_hbm.at[wid], idxt_v)
        pltpu.sync_copy(z_hbm.at[pl.ds(sid * rows_per_sub, rows_per_sub)],
                        agg_sh.at[pl.ds(sid * rows_per_sub, rows_per_sub)])
        plsc.subcore_barrier()

        def outer(t, _):
            for b in range(2):
                j = t * 2 + b

                @pl.when(t > 0)
                def _wait_prev_scatter():
                    pltpu.make_async_copy(
                        rows_v.at[b], agg_sh.at[idx_v.at[j]], ssem.at[b]).wait()

                pltpu.make_async_copy(
                    m_hbm.at[pl.ds(base + j * SCB, SCB)], rows_v.at[b],
                    msem.at[b]).start()
            for b in range(2):
                j = t * 2 + b
                pltpu.make_async_copy(
                    m_hbm.at[pl.ds(base + j * SCB, SCB)], rows_v.at[b],
                    msem.at[b]).wait()
                pltpu.make_async_copy(
                    rows_v.at[b], agg_sh.at[idx_v.at[j]],
                    ssem.at[b]).start(add=True)
            return 0

        lax.fori_loop(0, SFULL // 2, outer, 0)
        for b in range(2):
            pltpu.make_async_copy(
                rows_v.at[b], agg_sh.at[idx_v.at[b]], ssem.at[b]).wait()
        jlast = SFULL - 1
        pltpu.sync_copy(m_hbm.at[pl.ds(base + jlast * SCB, SCB)], rows_v.at[0])
        pltpu.sync_copy(rows_v.at[0], agg_sh.at[idx_v.at[jlast]], add=True)
        pltpu.sync_copy(m_hbm.at[pl.ds(base + SFULL * SCB, STAIL)], rowst_v)
        pltpu.sync_copy(rowst_v, agg_sh.at[idxt_v], add=True)
        plsc.subcore_barrier()
        pltpu.sync_copy(agg_sh.at[pl.ds(sid * rows_per_sub, rows_per_sub)],
                        out_hbm.at[cid, pl.ds(sid * rows_per_sub, rows_per_sub)])

    return k(m, ei_main, ei_tail, zeros_nh)


def _tc_edge_mlp(x_j, edge_f, W1a, W1b, b1, W2, b2):
    BE = 2000

    def body(xj_ref, f_ref, w1a, w1b, b1r, w2, b2r, out_ref):
        xjb = xj_ref[...].astype(jnp.bfloat16)
        h = jnp.dot(xjb, w1a[...], preferred_element_type=jnp.float32)
        h = h + jnp.dot(f_ref[...], w1b[...], preferred_element_type=jnp.float32)
        h = h + b1r[...]
        h = h * jax.nn.sigmoid(h)
        hb = h.astype(jnp.bfloat16)
        mm = jnp.dot(hb, w2[...], preferred_element_type=jnp.float32) + b2r[...]
        out_ref[...] = mm * jax.nn.sigmoid(mm)

    return pl.pallas_call(
        body,
        grid=(EH // BE,),
        in_specs=[
            pl.BlockSpec((BE, ATOM_DIM), lambda i: (i, 0)),
            pl.BlockSpec((BE, EDGE_DIM), lambda i: (i, 0)),
            pl.BlockSpec((ATOM_DIM, HIDDEN), lambda i: (0, 0)),
            pl.BlockSpec((EDGE_DIM, HIDDEN), lambda i: (0, 0)),
            pl.BlockSpec((1, HIDDEN), lambda i: (0, 0)),
            pl.BlockSpec((HIDDEN, HIDDEN), lambda i: (0, 0)),
            pl.BlockSpec((1, HIDDEN), lambda i: (0, 0)),
        ],
        out_specs=pl.BlockSpec((BE, HIDDEN), lambda i: (i, 0)),
        out_shape=jax.ShapeDtypeStruct((EH, HIDDEN), jnp.float32),
    )(x_j, edge_f, W1a, W1b, b1.reshape(1, HIDDEN), W2, b2.reshape(1, HIDDEN))


def _tc_final(x, aggs, Wl, bl):
    BN = 1000

    def body(x_ref, a0_ref, a1_ref, a2_ref, a3_ref, wl, blr, out_ref):
        a = ((a0_ref[...] + a1_ref[...]) + (a2_ref[...] + a3_ref[...]))
        t = x_ref[...] + jnp.dot(a, wl[...], preferred_element_type=jnp.float32)
        t = t + blr[...]
        out_ref[...] = t * jax.nn.sigmoid(t)

    agg_spec = pl.BlockSpec((BN, HIDDEN), lambda i: (i, 0))
    return pl.pallas_call(
        body,
        grid=(N_NODES // BN,),
        in_specs=[
            pl.BlockSpec((BN, ATOM_DIM), lambda i: (i, 0)),
            agg_spec, agg_spec, agg_spec, agg_spec,
            pl.BlockSpec((HIDDEN, ATOM_DIM), lambda i: (0, 0)),
            pl.BlockSpec((1, ATOM_DIM), lambda i: (0, 0)),
        ],
        out_specs=pl.BlockSpec((BN, ATOM_DIM), lambda i: (i, 0)),
        out_shape=jax.ShapeDtypeStruct((N_NODES, ATOM_DIM), jnp.float32),
    )(x, aggs[0], aggs[1], aggs[2], aggs[3], Wl, bl.reshape(1, ATOM_DIM))


def kernel(x, edge_i, edge_j, edge_f, W1, b1, W2, b2, Wl, bl):
    edge_i = edge_i.astype(jnp.int32)
    edge_j = edge_j.astype(jnp.int32)
    W1a = W1[:ATOM_DIM].astype(jnp.bfloat16)
    W1b = W1[ATOM_DIM:].astype(jnp.bfloat16)
    W2b = W2.astype(jnp.bfloat16)
    zeros = jnp.zeros((AGG_ROWS, HIDDEN), jnp.float32)

    ei_mains, ei_tails, efs, xjs = [], [], [], []
    for h in range(NH):
        ej3 = edge_j[h * EH:(h + 1) * EH].reshape(NW, GCHUNKS, GCB)
        ei2 = edge_i[h * EH:(h + 1) * EH].reshape(NW, EPW)
        ei_mains.append(ei2[:, :SFULL * SCB].reshape(NW, SFULL, SCB))
        ei_tails.append(ei2[:, SFULL * SCB:])
        efs.append(edge_f[h * EH:(h + 1) * EH].astype(jnp.bfloat16))
        xjs.append(_sc_gather(x, ej3))
    aggs = []
    for h in range(NH):
        m = _tc_edge_mlp(xjs[h], efs[h], W1a, W1b, b1, W2b, b2)
        agg2 = _sc_scatter_add(m, ei_mains[h], ei_tails[h], zeros)
        aggs.append(agg2[0, :N_NODES])
        aggs.append(agg2[1, :N_NODES])
    return _tc_final(x, aggs, Wl, bl)


# speedup vs baseline: 1.0151x; 1.0151x over previous
import functools

import jax
import jax.numpy as jnp
from jax import lax
from jax.experimental import pallas as pl
from jax.experimental.pallas import tpu as pltpu
from jax.experimental.pallas import tpu_sc as plsc

N_NODES = 10000
N_EDGES = 320000
ATOM_DIM = 128
EDGE_DIM = 16
HIDDEN = 128

NC = 2
NS = 16
NW = NC * NS
EPW = N_EDGES // NW
CB = 80
CHUNKS = EPW // CB
KB = 5
OUTER = CHUNKS // KB

SCB = 128
SFULL = 78
STAIL = EPW - SFULL * SCB
SOUTER = SFULL // 2

AGG_ROWS = 10240


def _sc_gather(xw, ej3):
    mesh = plsc.VectorSubcoreMesh(core_axis_name="c", subcore_axis_name="s")

    @functools.partial(
        pl.kernel, mesh=mesh,
        out_type=jax.ShapeDtypeStruct((N_EDGES, ATOM_DIM), jnp.float32),
        scratch_types=[
            pltpu.VMEM((CHUNKS, CB), jnp.int32),
            pltpu.VMEM((KB, CB, ATOM_DIM), jnp.float32),
            pltpu.SemaphoreType.DMA((KB,)),
            pltpu.SemaphoreType.DMA((KB,)),
        ],
    )
    def k(x_hbm, ej_hbm, out_hbm, idx_v, rows_v, gsem, wsem):
        wid = lax.axis_index("s") * NC + lax.axis_index("c")
        base = wid * EPW
        pltpu.sync_copy(ej_hbm.at[wid], idx_v)

        def outer(t, _):
            for b in range(KB):
                j = t * KB + b

                @pl.when(t > 0)
                def _wait_prev_write():
                    pltpu.make_async_copy(
                        rows_v.at[b], out_hbm.at[pl.ds(base + j * CB, CB)],
                        wsem.at[b]).wait()

                pltpu.make_async_copy(
                    x_hbm.at[idx_v.at[j]], rows_v.at[b], gsem.at[b]).start()
            for b in range(KB):
                j = t * KB + b
                pltpu.make_async_copy(
                    x_hbm.at[idx_v.at[j]], rows_v.at[b], gsem.at[b]).wait()
                pltpu.make_async_copy(
                    rows_v.at[b], out_hbm.at[pl.ds(base + j * CB, CB)],
                    wsem.at[b]).start()
            return 0

        lax.fori_loop(0, OUTER, outer, 0)
        for b in range(KB):
            pltpu.make_async_copy(
                rows_v.at[b], out_hbm.at[pl.ds(base + b * CB, CB)],
                wsem.at[b]).wait()

    return k(xw, ej3)


def _sc_scatter_add(m, ei_main, ei_tail, zeros_nh):
    mesh = plsc.VectorSubcoreMesh(core_axis_name="c", subcore_axis_name="s")
    rows_per_sub = AGG_ROWS // NS

    @functools.partial(
        pl.kernel, mesh=mesh,
        out_type=jax.ShapeDtypeStruct((NC, AGG_ROWS, HIDDEN), jnp.float32),
        scratch_types=[
            pltpu.VMEM((SFULL, SCB), jnp.int32),
            pltpu.VMEM((STAIL,), jnp.int32),
            pltpu.VMEM((2, SCB, HIDDEN), jnp.float32),
            pltpu.VMEM((STAIL, HIDDEN), jnp.float32),
            pltpu.VMEM_SHARED((AGG_ROWS, HIDDEN), jnp.float32),
            pltpu.SemaphoreType.DMA((2,)),
            pltpu.SemaphoreType.DMA((2,)),
        ],
    )
    def k(m_hbm, ei_hbm, eit_hbm, z_hbm, out_hbm,
          idx_v, idxt_v, rows_v, rowst_v, agg_sh, msem, ssem):
        cid = lax.axis_index("c")
        sid = lax.axis_index("s")
        wid = sid * NC + cid
        base = wid * EPW
        pltpu.sync_copy(ei_hbm.at[wid], idx_v)
        pltpu.sync_copy(eit_hbm.at[wid], idxt_v)
        pltpu.sync_copy(z_hbm.at[pl.ds(sid * rows_per_sub, rows_per_sub)],
                        agg_sh.at[pl.ds(sid * rows_per_sub, rows_per_sub)])
        plsc.subcore_barrier()

        def outer(t, _):
            for b in range(2):
                j = t * 2 + b

                @pl.when(t > 0)
                def _wait_prev_scatter():
                    pltpu.make_async_copy(
                        rows_v.at[b], agg_sh.at[idx_v.at[j]], ssem.at[b]).wait()

                pltpu.make_async_copy(
                    m_hbm.at[pl.ds(base + j * SCB, SCB)], rows_v.at[b],
                    msem.at[b]).start()
            for b in range(2):
                j = t * 2 + b
                pltpu.make_async_copy(
                    m_hbm.at[pl.ds(base + j * SCB, SCB)], rows_v.at[b],
                    msem.at[b]).wait()
                pltpu.make_async_copy(
                    rows_v.at[b], agg_sh.at[idx_v.at[j]],
                    ssem.at[b]).start(add=True)
            return 0

        lax.fori_loop(0, SOUTER, outer, 0)
        for b in range(2):
            pltpu.make_async_copy(
                rows_v.at[b], agg_sh.at[idx_v.at[b]], ssem.at[b]).wait()
        pltpu.sync_copy(m_hbm.at[pl.ds(base + SFULL * SCB, STAIL)], rowst_v)
        pltpu.sync_copy(rowst_v, agg_sh.at[idxt_v], add=True)
        plsc.subcore_barrier()
        pltpu.sync_copy(agg_sh.at[pl.ds(sid * rows_per_sub, rows_per_sub)],
                        out_hbm.at[cid, pl.ds(sid * rows_per_sub, rows_per_sub)])

    return k(m, ei_main, ei_tail, zeros_nh)


def _tc_edge_mlp(x_j, edge_f, W1a, W1b, b1, W2, b2):
    BE = 2560

    def body(xj_ref, f_ref, w1a, w1b, b1r, w2, b2r, out_ref):
        xjb = xj_ref[...].astype(jnp.bfloat16)
        h = jnp.dot(xjb, w1a[...], preferred_element_type=jnp.float32)
        fb = f_ref[...].astype(jnp.bfloat16)
        h = h + jnp.dot(fb, w1b[...], preferred_element_type=jnp.float32)
        h = h + b1r[...]
        h = h * jax.nn.sigmoid(h)
        hb = h.astype(jnp.bfloat16)
        mm = jnp.dot(hb, w2[...], preferred_element_type=jnp.float32) + b2r[...]
        out_ref[...] = mm * jax.nn.sigmoid(mm)

    return pl.pallas_call(
        body,
        grid=(N_EDGES // BE,),
        in_specs=[
            pl.BlockSpec((BE, ATOM_DIM), lambda i: (i, 0)),
            pl.BlockSpec((BE, EDGE_DIM), lambda i: (i, 0)),
            pl.BlockSpec((ATOM_DIM, HIDDEN), lambda i: (0, 0)),
            pl.BlockSpec((EDGE_DIM, HIDDEN), lambda i: (0, 0)),
            pl.BlockSpec((1, HIDDEN), lambda i: (0, 0)),
            pl.BlockSpec((HIDDEN, HIDDEN), lambda i: (0, 0)),
            pl.BlockSpec((1, HIDDEN), lambda i: (0, 0)),
        ],
        out_specs=pl.BlockSpec((BE, HIDDEN), lambda i: (i, 0)),
        out_shape=jax.ShapeDtypeStruct((N_EDGES, HIDDEN), jnp.float32),
    )(x_j, edge_f, W1a, W1b, b1.reshape(1, HIDDEN), W2, b2.reshape(1, HIDDEN))


def _tc_final(x, a0, a1, Wl, bl):
    BN = 1000

    def body(x_ref, a0_ref, a1_ref, wl, blr, out_ref):
        a = a0_ref[...] + a1_ref[...]
        t = x_ref[...] + jnp.dot(a, wl[...], preferred_element_type=jnp.float32)
        t = t + blr[...]
        out_ref[...] = t * jax.nn.sigmoid(t)

    return pl.pallas_call(
        body,
        grid=(N_NODES // BN,),
        in_specs=[
            pl.BlockSpec((BN, ATOM_DIM), lambda i: (i, 0)),
            pl.BlockSpec((BN, HIDDEN), lambda i: (i, 0)),
            pl.BlockSpec((BN, HIDDEN), lambda i: (i, 0)),
            pl.BlockSpec((HIDDEN, ATOM_DIM), lambda i: (0, 0)),
            pl.BlockSpec((1, ATOM_DIM), lambda i: (0, 0)),
        ],
        out_specs=pl.BlockSpec((BN, ATOM_DIM), lambda i: (i, 0)),
        out_shape=jax.ShapeDtypeStruct((N_NODES, ATOM_DIM), jnp.float32),
    )(x, a0, a1, Wl, bl.reshape(1, ATOM_DIM))


def kernel(x, edge_i, edge_j, edge_f, W1, b1, W2, b2, Wl, bl):
    ei2 = edge_i.astype(jnp.int32).reshape(NW, EPW)
    ei_main = ei2[:, :SFULL * SCB].reshape(NW, SFULL, SCB)
    ei_tail = ei2[:, SFULL * SCB:]
    edge_j = edge_j.astype(jnp.int32).reshape(NW, CHUNKS, CB)
    x_j = _sc_gather(x, edge_j)
    m = _tc_edge_mlp(x_j, edge_f,
                     W1[:ATOM_DIM].astype(jnp.bfloat16),
                     W1[ATOM_DIM:].astype(jnp.bfloat16),
                     b1, W2.astype(jnp.bfloat16), b2)
    zeros = jnp.zeros((AGG_ROWS, HIDDEN), jnp.float32)
    agg2 = _sc_scatter_add(m, ei_main, ei_tail, zeros)
    return _tc_final(x, agg2[0], agg2[1], Wl, bl)


# speedup vs baseline: 1.0875x; 1.0714x over previous
import functools

import jax
import jax.numpy as jnp
from jax import lax
from jax.experimental import pallas as pl
from jax.experimental.pallas import tpu as pltpu
from jax.experimental.pallas import tpu_sc as plsc

N_NODES = 10000
N_EDGES = 320000
ATOM_DIM = 128
EDGE_DIM = 16
HIDDEN = 128

NC = 2
NS = 16
NW = NC * NS
EPW = N_EDGES // NW
CB = 80
CHUNKS = EPW // CB
KB = 5
OUTER = CHUNKS // KB

SCB = 128
SFULL = 78
STAIL = EPW - SFULL * SCB
SOUTER = SFULL // 2

AGG_ROWS = 10240


def _sc_gather(xw, ej3):
    mesh = plsc.VectorSubcoreMesh(core_axis_name="c", subcore_axis_name="s")

    @functools.partial(
        pl.kernel, mesh=mesh,
        out_type=jax.ShapeDtypeStruct((N_EDGES, ATOM_DIM), jnp.float32),
        scratch_types=[
            pltpu.VMEM((CHUNKS, CB), jnp.int32),
            pltpu.VMEM((KB, CB, ATOM_DIM), jnp.float32),
            pltpu.SemaphoreType.DMA((KB,)),
            pltpu.SemaphoreType.DMA((KB,)),
        ],
    )
    def k(x_hbm, ej_hbm, out_hbm, idx_v, rows_v, gsem, wsem):
        wid = lax.axis_index("s") * NC + lax.axis_index("c")
        base = wid * EPW
        pltpu.sync_copy(ej_hbm.at[wid], idx_v)

        def outer(t, _):
            for b in range(KB):
                j = t * KB + b

                @pl.when(t > 0)
                def _wait_prev_write():
                    pltpu.make_async_copy(
                        rows_v.at[b], out_hbm.at[pl.ds(base + j * CB, CB)],
                        wsem.at[b]).wait()

                pltpu.make_async_copy(
                    x_hbm.at[idx_v.at[j]], rows_v.at[b], gsem.at[b]).start()
            for b in range(KB):
                j = t * KB + b
                pltpu.make_async_copy(
                    x_hbm.at[idx_v.at[j]], rows_v.at[b], gsem.at[b]).wait()
                pltpu.make_async_copy(
                    rows_v.at[b], out_hbm.at[pl.ds(base + j * CB, CB)],
                    wsem.at[b]).start()
            return 0

        lax.fori_loop(0, OUTER, outer, 0)
        for b in range(KB):
            pltpu.make_async_copy(
                rows_v.at[b], out_hbm.at[pl.ds(base + b * CB, CB)],
                wsem.at[b]).wait()

    return k(xw, ej3)


def _sc_scatter_add(m, ei_main, ei_tail, zeros_nh):
    mesh = plsc.VectorSubcoreMesh(core_axis_name="c", subcore_axis_name="s")
    rows_per_sub = AGG_ROWS // NS

    @functools.partial(
        pl.kernel, mesh=mesh,
        out_type=jax.ShapeDtypeStruct((NC, AGG_ROWS, HIDDEN), jnp.float32),
        scratch_types=[
            pltpu.VMEM((SFULL, SCB), jnp.int32),
            pltpu.VMEM((STAIL,), jnp.int32),
            pltpu.VMEM((2, SCB, HIDDEN), jnp.float32),
            pltpu.VMEM((STAIL, HIDDEN), jnp.float32),
            pltpu.VMEM_SHARED((AGG_ROWS, HIDDEN), jnp.float32),
            pltpu.SemaphoreType.DMA((2,)),
            pltpu.SemaphoreType.DMA((2,)),
        ],
    )
    def k(m_hbm, ei_hbm, eit_hbm, z_hbm, out_hbm,
          idx_v, idxt_v, rows_v, rowst_v, agg_sh, msem, ssem):
        cid = lax.axis_index("c")
        sid = lax.axis_index("s")
        wid = sid * NC + cid
        base = wid * EPW
        pltpu.sync_copy(ei_hbm.at[wid], idx_v)
        pltpu.sync_copy(eit_hbm.at[wid], idxt_v)
        pltpu.sync_copy(z_hbm.at[pl.ds(sid * rows_per_sub, rows_per_sub)],
                        agg_sh.at[pl.ds(sid * rows_per_sub, rows_per_sub)])
        plsc.subcore_barrier()

        def outer(t, _):
            for b in range(2):
                j = t * 2 + b

                @pl.when(t > 0)
                def _wait_prev_scatter():
                    pltpu.make_async_copy(
                        rows_v.at[b], agg_sh.at[idx_v.at[j]], ssem.at[b]).wait()

                pltpu.make_async_copy(
                    m_hbm.at[pl.ds(base + j * SCB, SCB)], rows_v.at[b],
                    msem.at[b]).start()
            for b in range(2):
                j = t * 2 + b
                pltpu.make_async_copy(
                    m_hbm.at[pl.ds(base + j * SCB, SCB)], rows_v.at[b],
                    msem.at[b]).wait()
                pltpu.make_async_copy(
                    rows_v.at[b], agg_sh.at[idx_v.at[j]],
                    ssem.at[b]).start(add=True)
            return 0

        lax.fori_loop(0, SOUTER, outer, 0)
        for b in range(2):
            pltpu.make_async_copy(
                rows_v.at[b], agg_sh.at[idx_v.at[b]], ssem.at[b]).wait()
        pltpu.sync_copy(m_hbm.at[pl.ds(base + SFULL * SCB, STAIL)], rowst_v)
        pltpu.sync_copy(rowst_v, agg_sh.at[idxt_v], add=True)
        plsc.subcore_barrier()
        pltpu.sync_copy(agg_sh.at[pl.ds(sid * rows_per_sub, rows_per_sub)],
                        out_hbm.at[cid, pl.ds(sid * rows_per_sub, rows_per_sub)])

    return k(m, ei_main, ei_tail, zeros_nh)


def _tc_edge_mlp(x_j, edge_f, W1a, W1b, b1, W2, b2):
    BE = 2560

    def body(xj_ref, f_ref, w1a, w1b, b1r, w2, b2r, out_ref):
        xjb = xj_ref[...].astype(jnp.bfloat16)
        h = jnp.dot(xjb, w1a[...], preferred_element_type=jnp.float32)
        h = h + jnp.dot(f_ref[...], w1b[...], preferred_element_type=jnp.float32)
        h = h + b1r[...]
        h = h * jax.nn.sigmoid(h)
        hb = h.astype(jnp.bfloat16)
        mm = jnp.dot(hb, w2[...], preferred_element_type=jnp.float32) + b2r[...]
        out_ref[...] = mm * jax.nn.sigmoid(mm)

    return pl.pallas_call(
        body,
        grid=(N_EDGES // BE,),
        in_specs=[
            pl.BlockSpec((BE, ATOM_DIM), lambda i: (i, 0)),
            pl.BlockSpec((BE, EDGE_DIM), lambda i: (i, 0)),
            pl.BlockSpec((ATOM_DIM, HIDDEN), lambda i: (0, 0)),
            pl.BlockSpec((EDGE_DIM, HIDDEN), lambda i: (0, 0)),
            pl.BlockSpec((1, HIDDEN), lambda i: (0, 0)),
            pl.BlockSpec((HIDDEN, HIDDEN), lambda i: (0, 0)),
            pl.BlockSpec((1, HIDDEN), lambda i: (0, 0)),
        ],
        out_specs=pl.BlockSpec((BE, HIDDEN), lambda i: (i, 0)),
        out_shape=jax.ShapeDtypeStruct((N_EDGES, HIDDEN), jnp.float32),
    )(x_j, edge_f, W1a, W1b, b1.reshape(1, HIDDEN), W2, b2.reshape(1, HIDDEN))


def _tc_final(x, agg2, Wl, bl):
    BN = 1000

    def body(x_ref, a0_ref, a1_ref, wl, blr, out_ref):
        a = a0_ref[0] + a1_ref[0]
        t = x_ref[...] + jnp.dot(a, wl[...], preferred_element_type=jnp.float32)
        t = t + blr[...]
        out_ref[...] = t * jax.nn.sigmoid(t)

    return pl.pallas_call(
        body,
        grid=(N_NODES // BN,),
        in_specs=[
            pl.BlockSpec((BN, ATOM_DIM), lambda i: (i, 0)),
            pl.BlockSpec((1, BN, HIDDEN), lambda i: (0, i, 0)),
            pl.BlockSpec((1, BN, HIDDEN), lambda i: (1, i, 0)),
            pl.BlockSpec((HIDDEN, ATOM_DIM), lambda i: (0, 0)),
            pl.BlockSpec((1, ATOM_DIM), lambda i: (0, 0)),
        ],
        out_specs=pl.BlockSpec((BN, ATOM_DIM), lambda i: (i, 0)),
        out_shape=jax.ShapeDtypeStruct((N_NODES, ATOM_DIM), jnp.float32),
    )(x, agg2, agg2, Wl, bl.reshape(1, ATOM_DIM))


def kernel(x, edge_i, edge_j, edge_f, W1, b1, W2, b2, Wl, bl):
    ei2 = edge_i.astype(jnp.int32).reshape(NW, EPW)
    ei_main = ei2[:, :SFULL * SCB].reshape(NW, SFULL, SCB)
    ei_tail = ei2[:, SFULL * SCB:]
    edge_j = edge_j.astype(jnp.int32).reshape(NW, CHUNKS, CB)
    x_j = _sc_gather(x, edge_j)
    m = _tc_edge_mlp(x_j, edge_f.astype(jnp.bfloat16),
                     W1[:ATOM_DIM].astype(jnp.bfloat16),
                     W1[ATOM_DIM:].astype(jnp.bfloat16),
                     b1, W2.astype(jnp.bfloat16), b2)
    zeros = jnp.zeros((AGG_ROWS, HIDDEN), jnp.float32)
    agg2 = _sc_scatter_add(m, ei_main, ei_tail, zeros)
    return _tc_final(x, agg2, Wl, bl)


# speedup vs baseline: 1.0887x; 1.0011x over previous
import functools

import jax
import jax.numpy as jnp
from jax import lax
from jax.experimental import pallas as pl
from jax.experimental.pallas import tpu as pltpu
from jax.experimental.pallas import tpu_sc as plsc

N_NODES = 10000
N_EDGES = 320000
ATOM_DIM = 128
EDGE_DIM = 16
HIDDEN = 128

NC = 2
NS = 16
NW = NC * NS
EPW = N_EDGES // NW
GCB = 128
GFULL = 78
GTAIL = EPW - GFULL * GCB
KB = 6
GOUTER = GFULL // KB

SCB = 128
SFULL = 78
STAIL = EPW - SFULL * SCB
SOUTER = SFULL // 2

AGG_ROWS = 10240


def _sc_gather(xw, ej_main, ej_tail):
    mesh = plsc.VectorSubcoreMesh(core_axis_name="c", subcore_axis_name="s")

    @functools.partial(
        pl.kernel, mesh=mesh,
        out_type=jax.ShapeDtypeStruct((N_EDGES, ATOM_DIM), jnp.float32),
        scratch_types=[
            pltpu.VMEM((GFULL, GCB), jnp.int32),
            pltpu.VMEM((GTAIL,), jnp.int32),
            pltpu.VMEM((KB, GCB, ATOM_DIM), jnp.float32),
            pltpu.VMEM((GTAIL, ATOM_DIM), jnp.float32),
            pltpu.SemaphoreType.DMA((KB,)),
            pltpu.SemaphoreType.DMA((KB,)),
        ],
    )
    def k(x_hbm, ej_hbm, ejt_hbm, out_hbm,
          idx_v, idxt_v, rows_v, rowst_v, gsem, wsem):
        wid = lax.axis_index("s") * NC + lax.axis_index("c")
        base = wid * EPW
        pltpu.sync_copy(ej_hbm.at[wid], idx_v)
        pltpu.sync_copy(ejt_hbm.at[wid], idxt_v)

        def outer(t, _):
            for b in range(KB):
                j = t * KB + b

                @pl.when(t > 0)
                def _wait_prev_write():
                    pltpu.make_async_copy(
                        rows_v.at[b], out_hbm.at[pl.ds(base + j * GCB, GCB)],
                        wsem.at[b]).wait()

                pltpu.make_async_copy(
                    x_hbm.at[idx_v.at[j]], rows_v.at[b], gsem.at[b]).start()
            for b in range(KB):
                j = t * KB + b
                pltpu.make_async_copy(
                    x_hbm.at[idx_v.at[j]], rows_v.at[b], gsem.at[b]).wait()
                pltpu.make_async_copy(
                    rows_v.at[b], out_hbm.at[pl.ds(base + j * GCB, GCB)],
                    wsem.at[b]).start()
            return 0

        lax.fori_loop(0, GOUTER, outer, 0)
        pltpu.make_async_copy(
            x_hbm.at[idxt_v], rowst_v, gsem.at[0]).start()
        for b in range(KB):
            pltpu.make_async_copy(
                rows_v.at[b], out_hbm.at[pl.ds(base + b * GCB, GCB)],
                wsem.at[b]).wait()
        pltpu.make_async_copy(x_hbm.at[idxt_v], rowst_v, gsem.at[0]).wait()
        pltpu.sync_copy(rowst_v, out_hbm.at[pl.ds(base + GFULL * GCB, GTAIL)])

    return k(xw, ej_main, ej_tail)


def _sc_scatter_add(m, ei_main, ei_tail, zeros_nh):
    mesh = plsc.VectorSubcoreMesh(core_axis_name="c", subcore_axis_name="s")
    rows_per_sub = AGG_ROWS // NS

    @functools.partial(
        pl.kernel, mesh=mesh,
        out_type=jax.ShapeDtypeStruct((NC, AGG_ROWS, HIDDEN), jnp.float32),
        scratch_types=[
            pltpu.VMEM((SFULL, SCB), jnp.int32),
            pltpu.VMEM((STAIL,), jnp.int32),
            pltpu.VMEM((2, SCB, HIDDEN), jnp.float32),
            pltpu.VMEM((STAIL, HIDDEN), jnp.float32),
            pltpu.VMEM_SHARED((AGG_ROWS, HIDDEN), jnp.float32),
            pltpu.SemaphoreType.DMA((2,)),
            pltpu.SemaphoreType.DMA((2,)),
        ],
    )
    def k(m_hbm, ei_hbm, eit_hbm, z_hbm, out_hbm,
          idx_v, idxt_v, rows_v, rowst_v, agg_sh, msem, ssem):
        cid = lax.axis_index("c")
        sid = lax.axis_index("s")
        wid = sid * NC + cid
        base = wid * EPW
        pltpu.sync_copy(ei_hbm.at[wid], idx_v)
        pltpu.sync_copy(eit_hbm.at[wid], idxt_v)
        pltpu.sync_copy(z_hbm.at[pl.ds(sid * rows_per_sub, rows_per_sub)],
                        agg_sh.at[pl.ds(sid * rows_per_sub, rows_per_sub)])
        plsc.subcore_barrier()

        def outer(t, _):
            for b in range(2):
                j = t * 2 + b

                @pl.when(t > 0)
                def _wait_prev_scatter():
                    pltpu.make_async_copy(
                        rows_v.at[b], agg_sh.at[idx_v.at[j]], ssem.at[b]).wait()

                pltpu.make_async_copy(
                    m_hbm.at[pl.ds(base + j * SCB, SCB)], rows_v.at[b],
                    msem.at[b]).start()
            for b in range(2):
                j = t * 2 + b
                pltpu.make_async_copy(
                    m_hbm.at[pl.ds(base + j * SCB, SCB)], rows_v.at[b],
                    msem.at[b]).wait()
                pltpu.make_async_copy(
                    rows_v.at[b], agg_sh.at[idx_v.at[j]],
                    ssem.at[b]).start(add=True)
            return 0

        lax.fori_loop(0, SOUTER, outer, 0)
        for b in range(2):
            pltpu.make_async_copy(
                rows_v.at[b], agg_sh.at[idx_v.at[b]], ssem.at[b]).wait()
        pltpu.sync_copy(m_hbm.at[pl.ds(base + SFULL * SCB, STAIL)], rowst_v)
        pltpu.sync_copy(rowst_v, agg_sh.at[idxt_v], add=True)
        plsc.subcore_barrier()
        pltpu.sync_copy(agg_sh.at[pl.ds(sid * rows_per_sub, rows_per_sub)],
                        out_hbm.at[cid, pl.ds(sid * rows_per_sub, rows_per_sub)])

    return k(m, ei_main, ei_tail, zeros_nh)


def _tc_edge_mlp(x_j, edge_f, W1a, W1b, b1, W2, b2):
    BE = 2560

    def body(xj_ref, f_ref, w1a, w1b, b1r, w2, b2r, out_ref):
        xjb = xj_ref[...].astype(jnp.bfloat16)
        h = jnp.dot(xjb, w1a[...], preferred_element_type=jnp.float32)
        h = h + jnp.dot(f_ref[...], w1b[...], preferred_element_type=jnp.float32)
        h = h + b1r[...]
        h = h * jax.nn.sigmoid(h)
        hb = h.astype(jnp.bfloat16)
        mm = jnp.dot(hb, w2[...], preferred_element_type=jnp.float32) + b2r[...]
        out_ref[...] = mm * jax.nn.sigmoid(mm)

    return pl.pallas_call(
        body,
        grid=(N_EDGES // BE,),
        in_specs=[
            pl.BlockSpec((BE, ATOM_DIM), lambda i: (i, 0)),
            pl.BlockSpec((BE, EDGE_DIM), lambda i: (i, 0)),
            pl.BlockSpec((ATOM_DIM, HIDDEN), lambda i: (0, 0)),
            pl.BlockSpec((EDGE_DIM, HIDDEN), lambda i: (0, 0)),
            pl.BlockSpec((1, HIDDEN), lambda i: (0, 0)),
            pl.BlockSpec((HIDDEN, HIDDEN), lambda i: (0, 0)),
            pl.BlockSpec((1, HIDDEN), lambda i: (0, 0)),
        ],
        out_specs=pl.BlockSpec((BE, HIDDEN), lambda i: (i, 0)),
        out_shape=jax.ShapeDtypeStruct((N_EDGES, HIDDEN), jnp.float32),
    )(x_j, edge_f, W1a, W1b, b1.reshape(1, HIDDEN), W2, b2.reshape(1, HIDDEN))


def _tc_final(x, agg2, Wl, bl):
    BN = 1000

    def body(x_ref, a0_ref, a1_ref, wl, blr, out_ref):
        a = a0_ref[0] + a1_ref[0]
        t = x_ref[...] + jnp.dot(a, wl[...], preferred_element_type=jnp.float32)
        t = t + blr[...]
        out_ref[...] = t * jax.nn.sigmoid(t)

    return pl.pallas_call(
        body,
        grid=(N_NODES // BN,),
        in_specs=[
            pl.BlockSpec((BN, ATOM_DIM), lambda i: (i, 0)),
            pl.BlockSpec((1, BN, HIDDEN), lambda i: (0, i, 0)),
            pl.BlockSpec((1, BN, HIDDEN), lambda i: (1, i, 0)),
            pl.BlockSpec((HIDDEN, ATOM_DIM), lambda i: (0, 0)),
            pl.BlockSpec((1, ATOM_DIM), lambda i: (0, 0)),
        ],
        out_specs=pl.BlockSpec((BN, ATOM_DIM), lambda i: (i, 0)),
        out_shape=jax.ShapeDtypeStruct((N_NODES, ATOM_DIM), jnp.float32),
    )(x, agg2, agg2, Wl, bl.reshape(1, ATOM_DIM))


def kernel(x, edge_i, edge_j, edge_f, W1, b1, W2, b2, Wl, bl):
    ei2 = edge_i.astype(jnp.int32).reshape(NW, EPW)
    ei_main = ei2[:, :SFULL * SCB].reshape(NW, SFULL, SCB)
    ei_tail = ei2[:, SFULL * SCB:]
    ej2 = edge_j.astype(jnp.int32).reshape(NW, EPW)
    ej_main = ej2[:, :GFULL * GCB].reshape(NW, GFULL, GCB)
    ej_tail = ej2[:, GFULL * GCB:]
    x_j = _sc_gather(x, ej_main, ej_tail)
    m = _tc_edge_mlp(x_j, edge_f.astype(jnp.bfloat16),
                     W1[:ATOM_DIM].astype(jnp.bfloat16),
                     W1[ATOM_DIM:].astype(jnp.bfloat16),
                     b1, W2.astype(jnp.bfloat16), b2)
    zeros = jnp.zeros((AGG_ROWS, HIDDEN), jnp.float32)
    agg2 = _sc_scatter_add(m, ei_main, ei_tail, zeros)
    return _tc_final(x, agg2, Wl, bl)


# speedup vs baseline: 1.1456x; 1.0522x over previous
import functools

import jax
import jax.numpy as jnp
from jax import lax
from jax.experimental import pallas as pl
from jax.experimental.pallas import tpu as pltpu
from jax.experimental.pallas import tpu_sc as plsc

N_NODES = 10000
N_EDGES = 320000
ATOM_DIM = 128
EDGE_DIM = 16
HIDDEN = 128

NC = 2
NS = 16
NW = NC * NS
EPW = N_EDGES // NW
GCB = 128
GFULL = 78
GTAIL = EPW - GFULL * GCB
KB = 6
GOUTER = GFULL // KB

SCB = 128
SFULL = 78
STAIL = EPW - SFULL * SCB
SOUTER = SFULL // 2

AGG_ROWS = 10240


def _sc_gather(xw, ej_main, ej_tail):
    mesh = plsc.VectorSubcoreMesh(core_axis_name="c", subcore_axis_name="s")

    @functools.partial(
        pl.kernel, mesh=mesh,
        out_type=jax.ShapeDtypeStruct((N_EDGES, ATOM_DIM), jnp.float32),
        scratch_types=[
            pltpu.VMEM((GFULL, GCB), jnp.int32),
            pltpu.VMEM((GTAIL,), jnp.int32),
            pltpu.VMEM((KB, GCB, ATOM_DIM), jnp.float32),
            pltpu.VMEM((GTAIL, ATOM_DIM), jnp.float32),
            pltpu.SemaphoreType.DMA((KB,)),
            pltpu.SemaphoreType.DMA((KB,)),
        ],
    )
    def k(x_hbm, ej_hbm, ejt_hbm, out_hbm,
          idx_v, idxt_v, rows_v, rowst_v, gsem, wsem):
        wid = lax.axis_index("s") * NC + lax.axis_index("c")
        base = wid * EPW
        pltpu.sync_copy(ej_hbm.at[wid], idx_v)
        pltpu.sync_copy(ejt_hbm.at[wid], idxt_v)

        def outer(t, _):
            for b in range(KB):
                j = t * KB + b

                @pl.when(t > 0)
                def _wait_prev_write():
                    pltpu.make_async_copy(
                        rows_v.at[b], out_hbm.at[pl.ds(base + j * GCB, GCB)],
                        wsem.at[b]).wait()

                pltpu.make_async_copy(
                    x_hbm.at[idx_v.at[j]], rows_v.at[b], gsem.at[b]).start()
            for b in range(KB):
                j = t * KB + b
                pltpu.make_async_copy(
                    x_hbm.at[idx_v.at[j]], rows_v.at[b], gsem.at[b]).wait()
                pltpu.make_async_copy(
                    rows_v.at[b], out_hbm.at[pl.ds(base + j * GCB, GCB)],
                    wsem.at[b]).start()
            return 0

        lax.fori_loop(0, GOUTER, outer, 0)
        pltpu.make_async_copy(
            x_hbm.at[idxt_v], rowst_v, gsem.at[0]).start()
        for b in range(KB):
            pltpu.make_async_copy(
                rows_v.at[b], out_hbm.at[pl.ds(base + b * GCB, GCB)],
                wsem.at[b]).wait()
        pltpu.make_async_copy(x_hbm.at[idxt_v], rowst_v, gsem.at[0]).wait()
        pltpu.sync_copy(rowst_v, out_hbm.at[pl.ds(base + GFULL * GCB, GTAIL)])

    return k(xw, ej_main, ej_tail)


def _sc_scatter_add(m, ei_main, ei_tail, zeros_nh):
    mesh = plsc.VectorSubcoreMesh(core_axis_name="c", subcore_axis_name="s")
    rows_per_sub = AGG_ROWS // NS

    @functools.partial(
        pl.kernel, mesh=mesh,
        out_type=jax.ShapeDtypeStruct((NC, AGG_ROWS, HIDDEN), jnp.float32),
        scratch_types=[
            pltpu.VMEM((SFULL, SCB), jnp.int32),
            pltpu.VMEM((STAIL,), jnp.int32),
            pltpu.VMEM((2, SCB, HIDDEN), jnp.float32),
            pltpu.VMEM((STAIL, HIDDEN), jnp.float32),
            pltpu.VMEM_SHARED((AGG_ROWS, HIDDEN), jnp.float32),
            pltpu.SemaphoreType.DMA((2,)),
            pltpu.SemaphoreType.DMA((2,)),
        ],
    )
    def k(m_hbm, ei_hbm, eit_hbm, z_hbm, out_hbm,
          idx_v, idxt_v, rows_v, rowst_v, agg_sh, msem, ssem):
        cid = lax.axis_index("c")
        sid = lax.axis_index("s")
        wid = sid * NC + cid
        base = wid * EPW
        pltpu.sync_copy(ei_hbm.at[wid], idx_v)
        pltpu.sync_copy(eit_hbm.at[wid], idxt_v)
        pltpu.sync_copy(z_hbm.at[pl.ds(sid * rows_per_sub, rows_per_sub)],
                        agg_sh.at[pl.ds(sid * rows_per_sub, rows_per_sub)])
        plsc.subcore_barrier()

        def outer(t, _):
            for b in range(2):
                j = t * 2 + b

                @pl.when(t > 0)
                def _wait_prev_scatter():
                    pltpu.make_async_copy(
                        rows_v.at[b], agg_sh.at[idx_v.at[j]], ssem.at[b]).wait()

                pltpu.make_async_copy(
                    m_hbm.at[pl.ds(base + j * SCB, SCB)], rows_v.at[b],
                    msem.at[b]).start()
            for b in range(2):
                j = t * 2 + b
                pltpu.make_async_copy(
                    m_hbm.at[pl.ds(base + j * SCB, SCB)], rows_v.at[b],
                    msem.at[b]).wait()
                pltpu.make_async_copy(
                    rows_v.at[b], agg_sh.at[idx_v.at[j]],
                    ssem.at[b]).start(add=True)
            return 0

        lax.fori_loop(0, SOUTER, outer, 0)
        for b in range(2):
            pltpu.make_async_copy(
                rows_v.at[b], agg_sh.at[idx_v.at[b]], ssem.at[b]).wait()
        pltpu.sync_copy(m_hbm.at[pl.ds(base + SFULL * SCB, STAIL)], rowst_v)
        pltpu.sync_copy(rowst_v, agg_sh.at[idxt_v], add=True)
        plsc.subcore_barrier()
        pltpu.sync_copy(agg_sh.at[pl.ds(sid * rows_per_sub, rows_per_sub)],
                        out_hbm.at[cid, pl.ds(sid * rows_per_sub, rows_per_sub)])

    return k(m, ei_main, ei_tail, zeros_nh)


def _tc_edge_mlp(x_j, edge_f, W1a, W1b, b1, W2, b2):
    BE = 4000

    def body(xj_ref, f_ref, w1a, w1b, b1r, w2, b2r, out_ref):
        xjb = xj_ref[...].astype(jnp.bfloat16)
        h = jnp.dot(xjb, w1a[...], preferred_element_type=jnp.float32)
        h = h + jnp.dot(f_ref[...], w1b[...], preferred_element_type=jnp.float32)
        h = h + b1r[...]
        h = h * jax.nn.sigmoid(h)
        hb = h.astype(jnp.bfloat16)
        mm = jnp.dot(hb, w2[...], preferred_element_type=jnp.float32) + b2r[...]
        out_ref[...] = mm * jax.nn.sigmoid(mm)

    return pl.pallas_call(
        body,
        grid=(N_EDGES // BE,),
        in_specs=[
            pl.BlockSpec((BE, ATOM_DIM), lambda i: (i, 0)),
            pl.BlockSpec((BE, EDGE_DIM), lambda i: (i, 0)),
            pl.BlockSpec((ATOM_DIM, HIDDEN), lambda i: (0, 0)),
            pl.BlockSpec((EDGE_DIM, HIDDEN), lambda i: (0, 0)),
            pl.BlockSpec((1, HIDDEN), lambda i: (0, 0)),
            pl.BlockSpec((HIDDEN, HIDDEN), lambda i: (0, 0)),
            pl.BlockSpec((1, HIDDEN), lambda i: (0, 0)),
        ],
        out_specs=pl.BlockSpec((BE, HIDDEN), lambda i: (i, 0)),
        out_shape=jax.ShapeDtypeStruct((N_EDGES, HIDDEN), jnp.float32),
    )(x_j, edge_f, W1a, W1b, b1.reshape(1, HIDDEN), W2, b2.reshape(1, HIDDEN))


def _tc_final(x, agg2, Wl, bl):
    BN = 1000

    def body(x_ref, a0_ref, a1_ref, wl, blr, out_ref):
        a = a0_ref[0] + a1_ref[0]
        t = x_ref[...] + jnp.dot(a, wl[...], preferred_element_type=jnp.float32)
        t = t + blr[...]
        out_ref[...] = t * jax.nn.sigmoid(t)

    return pl.pallas_call(
        body,
        grid=(N_NODES // BN,),
        in_specs=[
            pl.BlockSpec((BN, ATOM_DIM), lambda i: (i, 0)),
            pl.BlockSpec((1, BN, HIDDEN), lambda i: (0, i, 0)),
            pl.BlockSpec((1, BN, HIDDEN), lambda i: (1, i, 0)),
            pl.BlockSpec((HIDDEN, ATOM_DIM), lambda i: (0, 0)),
            pl.BlockSpec((1, ATOM_DIM), lambda i: (0, 0)),
        ],
        out_specs=pl.BlockSpec((BN, ATOM_DIM), lambda i: (i, 0)),
        out_shape=jax.ShapeDtypeStruct((N_NODES, ATOM_DIM), jnp.float32),
    )(x, agg2, agg2, Wl, bl.reshape(1, ATOM_DIM))


def kernel(x, edge_i, edge_j, edge_f, W1, b1, W2, b2, Wl, bl):
    ei2 = edge_i.astype(jnp.int32).reshape(NW, EPW)
    ei_main = ei2[:, :SFULL * SCB].reshape(NW, SFULL, SCB)
    ei_tail = ei2[:, SFULL * SCB:]
    ej2 = edge_j.astype(jnp.int32).reshape(NW, EPW)
    ej_main = ej2[:, :GFULL * GCB].reshape(NW, GFULL, GCB)
    ej_tail = ej2[:, GFULL * GCB:]
    x_j = _sc_gather(x, ej_main, ej_tail)
    m = _tc_edge_mlp(x_j, edge_f.astype(jnp.bfloat16),
                     W1[:ATOM_DIM].astype(jnp.bfloat16),
                     W1[ATOM_DIM:].astype(jnp.bfloat16),
                     b1, W2.astype(jnp.bfloat16), b2)
    zeros = jnp.zeros((AGG_ROWS, HIDDEN), jnp.float32)
    agg2 = _sc_scatter_add(m, ei_main, ei_tail, zeros)
    return _tc_final(x, agg2, Wl, bl)


# speedup vs baseline: 1.2141x; 1.0598x over previous
import functools

import jax
import jax.numpy as jnp
from jax import lax
from jax.experimental import pallas as pl
from jax.experimental.pallas import tpu as pltpu
from jax.experimental.pallas import tpu_sc as plsc

N_NODES = 10000
N_EDGES = 320000
ATOM_DIM = 128
EDGE_DIM = 16
HIDDEN = 128

NC = 2
NS = 16
NW = NC * NS
EPW = N_EDGES // NW
GCB = 128
GFULL = 78
GTAIL = EPW - GFULL * GCB
KB = 6
GOUTER = GFULL // KB

SCB = 128
SFULL = 78
STAIL = EPW - SFULL * SCB
SOUTER = SFULL // 2

AGG_ROWS = 10240


def _sc_gather(xw, ej_main, ej_tail):
    mesh = plsc.VectorSubcoreMesh(core_axis_name="c", subcore_axis_name="s")

    @functools.partial(
        pl.kernel, mesh=mesh,
        out_type=jax.ShapeDtypeStruct((N_EDGES, ATOM_DIM), jnp.float32),
        scratch_types=[
            pltpu.VMEM((GFULL, GCB), jnp.int32),
            pltpu.VMEM((GTAIL,), jnp.int32),
            pltpu.VMEM((KB, GCB, ATOM_DIM), jnp.float32),
            pltpu.VMEM((GTAIL, ATOM_DIM), jnp.float32),
            pltpu.SemaphoreType.DMA((KB,)),
            pltpu.SemaphoreType.DMA((KB,)),
        ],
    )
    def k(x_hbm, ej_hbm, ejt_hbm, out_hbm,
          idx_v, idxt_v, rows_v, rowst_v, gsem, wsem):
        wid = lax.axis_index("s") * NC + lax.axis_index("c")
        base = wid * EPW
        pltpu.sync_copy(ej_hbm.at[wid], idx_v)
        pltpu.sync_copy(ejt_hbm.at[wid], idxt_v)

        def outer(t, _):
            for b in range(KB):
                j = t * KB + b

                @pl.when(t > 0)
                def _wait_prev_write():
                    pltpu.make_async_copy(
                        rows_v.at[b], out_hbm.at[pl.ds(base + j * GCB, GCB)],
                        wsem.at[b]).wait()

                pltpu.make_async_copy(
                    x_hbm.at[idx_v.at[j]], rows_v.at[b], gsem.at[b]).start()
            for b in range(KB):
                j = t * KB + b
                pltpu.make_async_copy(
                    x_hbm.at[idx_v.at[j]], rows_v.at[b], gsem.at[b]).wait()
                pltpu.make_async_copy(
                    rows_v.at[b], out_hbm.at[pl.ds(base + j * GCB, GCB)],
                    wsem.at[b]).start()
            return 0

        lax.fori_loop(0, GOUTER, outer, 0)
        pltpu.make_async_copy(
            x_hbm.at[idxt_v], rowst_v, gsem.at[0]).start()
        for b in range(KB):
            pltpu.make_async_copy(
                rows_v.at[b], out_hbm.at[pl.ds(base + b * GCB, GCB)],
                wsem.at[b]).wait()
        pltpu.make_async_copy(x_hbm.at[idxt_v], rowst_v, gsem.at[0]).wait()
        pltpu.sync_copy(rowst_v, out_hbm.at[pl.ds(base + GFULL * GCB, GTAIL)])

    return k(xw, ej_main, ej_tail)


def _sc_scatter_add(m, ei_main, ei_tail, zeros_nh):
    mesh = plsc.VectorSubcoreMesh(core_axis_name="c", subcore_axis_name="s")
    rows_per_sub = AGG_ROWS // NS

    @functools.partial(
        pl.kernel, mesh=mesh,
        out_type=jax.ShapeDtypeStruct((NC, AGG_ROWS, HIDDEN), jnp.float32),
        scratch_types=[
            pltpu.VMEM((SFULL, SCB), jnp.int32),
            pltpu.VMEM((STAIL,), jnp.int32),
            pltpu.VMEM((2, SCB, HIDDEN), jnp.float32),
            pltpu.VMEM((STAIL, HIDDEN), jnp.float32),
            pltpu.VMEM_SHARED((AGG_ROWS, HIDDEN), jnp.float32),
            pltpu.SemaphoreType.DMA((2,)),
            pltpu.SemaphoreType.DMA((2,)),
        ],
    )
    def k(m_hbm, ei_hbm, eit_hbm, z_hbm, out_hbm,
          idx_v, idxt_v, rows_v, rowst_v, agg_sh, msem, ssem):
        cid = lax.axis_index("c")
        sid = lax.axis_index("s")
        wid = sid * NC + cid
        base = wid * EPW
        pltpu.sync_copy(ei_hbm.at[wid], idx_v)
        pltpu.sync_copy(eit_hbm.at[wid], idxt_v)
        pltpu.sync_copy(z_hbm.at[pl.ds(sid * rows_per_sub, rows_per_sub)],
                        agg_sh.at[pl.ds(sid * rows_per_sub, rows_per_sub)])
        plsc.subcore_barrier()

        def outer(t, _):
            for b in range(2):
                j = t * 2 + b

                @pl.when(t > 0)
                def _wait_prev_scatter():
                    pltpu.make_async_copy(
                        rows_v.at[b], agg_sh.at[idx_v.at[j]], ssem.at[b]).wait()

                pltpu.make_async_copy(
                    m_hbm.at[pl.ds(base + j * SCB, SCB)], rows_v.at[b],
                    msem.at[b]).start()
            for b in range(2):
                j = t * 2 + b
                pltpu.make_async_copy(
                    m_hbm.at[pl.ds(base + j * SCB, SCB)], rows_v.at[b],
                    msem.at[b]).wait()
                pltpu.make_async_copy(
                    rows_v.at[b], agg_sh.at[idx_v.at[j]],
                    ssem.at[b]).start(add=True)
            return 0

        lax.fori_loop(0, SOUTER, outer, 0)
        for b in range(2):
            pltpu.make_async_copy(
                rows_v.at[b], agg_sh.at[idx_v.at[b]], ssem.at[b]).wait()
        pltpu.sync_copy(m_hbm.at[pl.ds(base + SFULL * SCB, STAIL)], rowst_v)
        pltpu.sync_copy(rowst_v, agg_sh.at[idxt_v], add=True)
        plsc.subcore_barrier()
        pltpu.sync_copy(agg_sh.at[pl.ds(sid * rows_per_sub, rows_per_sub)],
                        out_hbm.at[cid, pl.ds(sid * rows_per_sub, rows_per_sub)])

    return k(m, ei_main, ei_tail, zeros_nh)


def _tc_edge_mlp(x_j, edge_f, W1a, W1b, b1, W2, b2):
    BE = 8000

    def body(xj_ref, f_ref, w1a, w1b, b1r, w2, b2r, out_ref):
        xjb = xj_ref[...].astype(jnp.bfloat16)
        h = jnp.dot(xjb, w1a[...], preferred_element_type=jnp.float32)
        h = h + jnp.dot(f_ref[...], w1b[...], preferred_element_type=jnp.float32)
        h = h + b1r[...]
        h = h * jax.nn.sigmoid(h)
        hb = h.astype(jnp.bfloat16)
        mm = jnp.dot(hb, w2[...], preferred_element_type=jnp.float32) + b2r[...]
        out_ref[...] = mm * jax.nn.sigmoid(mm)

    return pl.pallas_call(
        body,
        grid=(N_EDGES // BE,),
        in_specs=[
            pl.BlockSpec((BE, ATOM_DIM), lambda i: (i, 0)),
            pl.BlockSpec((BE, EDGE_DIM), lambda i: (i, 0)),
            pl.BlockSpec((ATOM_DIM, HIDDEN), lambda i: (0, 0)),
            pl.BlockSpec((EDGE_DIM, HIDDEN), lambda i: (0, 0)),
            pl.BlockSpec((1, HIDDEN), lambda i: (0, 0)),
            pl.BlockSpec((HIDDEN, HIDDEN), lambda i: (0, 0)),
            pl.BlockSpec((1, HIDDEN), lambda i: (0, 0)),
        ],
        out_specs=pl.BlockSpec((BE, HIDDEN), lambda i: (i, 0)),
        out_shape=jax.ShapeDtypeStruct((N_EDGES, HIDDEN), jnp.float32),
    )(x_j, edge_f, W1a, W1b, b1.reshape(1, HIDDEN), W2, b2.reshape(1, HIDDEN))


def _tc_final(x, agg2, Wl, bl):
    BN = 1000

    def body(x_ref, a0_ref, a1_ref, wl, blr, out_ref):
        a = a0_ref[0] + a1_ref[0]
        t = x_ref[...] + jnp.dot(a, wl[...], preferred_element_type=jnp.float32)
        t = t + blr[...]
        out_ref[...] = t * jax.nn.sigmoid(t)

    return pl.pallas_call(
        body,
        grid=(N_NODES // BN,),
        in_specs=[
            pl.BlockSpec((BN, ATOM_DIM), lambda i: (i, 0)),
            pl.BlockSpec((1, BN, HIDDEN), lambda i: (0, i, 0)),
            pl.BlockSpec((1, BN, HIDDEN), lambda i: (1, i, 0)),
            pl.BlockSpec((HIDDEN, ATOM_DIM), lambda i: (0, 0)),
            pl.BlockSpec((1, ATOM_DIM), lambda i: (0, 0)),
        ],
        out_specs=pl.BlockSpec((BN, ATOM_DIM), lambda i: (i, 0)),
        out_shape=jax.ShapeDtypeStruct((N_NODES, ATOM_DIM), jnp.float32),
    )(x, agg2, agg2, Wl, bl.reshape(1, ATOM_DIM))


def kernel(x, edge_i, edge_j, edge_f, W1, b1, W2, b2, Wl, bl):
    ei2 = edge_i.astype(jnp.int32).reshape(NW, EPW)
    ei_main = ei2[:, :SFULL * SCB].reshape(NW, SFULL, SCB)
    ei_tail = ei2[:, SFULL * SCB:]
    ej2 = edge_j.astype(jnp.int32).reshape(NW, EPW)
    ej_main = ej2[:, :GFULL * GCB].reshape(NW, GFULL, GCB)
    ej_tail = ej2[:, GFULL * GCB:]
    x_j = _sc_gather(x, ej_main, ej_tail)
    m = _tc_edge_mlp(x_j, edge_f.astype(jnp.bfloat16),
                     W1[:ATOM_DIM].astype(jnp.bfloat16),
                     W1[ATOM_DIM:].astype(jnp.bfloat16),
                     b1, W2.astype(jnp.bfloat16), b2)
    zeros = jnp.zeros((AGG_ROWS, HIDDEN), jnp.float32)
    agg2 = _sc_scatter_add(m, ei_main, ei_tail, zeros)
    return _tc_final(x, agg2, Wl, bl)


# speedup vs baseline: 1.2370x; 1.0189x over previous
import functools

import jax
import jax.numpy as jnp
from jax import lax
from jax.experimental import pallas as pl
from jax.experimental.pallas import tpu as pltpu
from jax.experimental.pallas import tpu_sc as plsc

N_NODES = 10000
N_EDGES = 320000
ATOM_DIM = 128
EDGE_DIM = 16
HIDDEN = 128

NC = 2
NS = 16
NW = NC * NS
EPW = N_EDGES // NW
GCB = 128
GFULL = 78
GTAIL = EPW - GFULL * GCB
KB = 6
GOUTER = GFULL // KB

SCB = 128
SFULL = 78
STAIL = EPW - SFULL * SCB
SOUTER = SFULL // 2

AGG_ROWS = 10240


def _sc_gather(xw, ej_main, ej_tail):
    mesh = plsc.VectorSubcoreMesh(core_axis_name="c", subcore_axis_name="s")

    @functools.partial(
        pl.kernel, mesh=mesh,
        out_type=jax.ShapeDtypeStruct((N_EDGES, ATOM_DIM), jnp.float32),
        scratch_types=[
            pltpu.VMEM((GFULL, GCB), jnp.int32),
            pltpu.VMEM((GTAIL,), jnp.int32),
            pltpu.VMEM((KB, GCB, ATOM_DIM), jnp.float32),
            pltpu.VMEM((GTAIL, ATOM_DIM), jnp.float32),
            pltpu.SemaphoreType.DMA((KB,)),
            pltpu.SemaphoreType.DMA((KB,)),
        ],
    )
    def k(x_hbm, ej_hbm, ejt_hbm, out_hbm,
          idx_v, idxt_v, rows_v, rowst_v, gsem, wsem):
        wid = lax.axis_index("s") * NC + lax.axis_index("c")
        base = wid * EPW
        pltpu.sync_copy(ej_hbm.at[wid], idx_v)
        pltpu.sync_copy(ejt_hbm.at[wid], idxt_v)

        def outer(t, _):
            for b in range(KB):
                j = t * KB + b

                @pl.when(t > 0)
                def _wait_prev_write():
                    pltpu.make_async_copy(
                        rows_v.at[b], out_hbm.at[pl.ds(base + j * GCB, GCB)],
                        wsem.at[b]).wait()

                pltpu.make_async_copy(
                    x_hbm.at[idx_v.at[j]], rows_v.at[b], gsem.at[b]).start()
            for b in range(KB):
                j = t * KB + b
                pltpu.make_async_copy(
                    x_hbm.at[idx_v.at[j]], rows_v.at[b], gsem.at[b]).wait()
                pltpu.make_async_copy(
                    rows_v.at[b], out_hbm.at[pl.ds(base + j * GCB, GCB)],
                    wsem.at[b]).start()
            return 0

        lax.fori_loop(0, GOUTER, outer, 0)
        pltpu.make_async_copy(
            x_hbm.at[idxt_v], rowst_v, gsem.at[0]).start()
        for b in range(KB):
            pltpu.make_async_copy(
                rows_v.at[b], out_hbm.at[pl.ds(base + b * GCB, GCB)],
                wsem.at[b]).wait()
        pltpu.make_async_copy(x_hbm.at[idxt_v], rowst_v, gsem.at[0]).wait()
        pltpu.sync_copy(rowst_v, out_hbm.at[pl.ds(base + GFULL * GCB, GTAIL)])

    return k(xw, ej_main, ej_tail)


def _sc_scatter_add(m, ei_main, ei_tail, zeros_nh):
    mesh = plsc.VectorSubcoreMesh(core_axis_name="c", subcore_axis_name="s")
    rows_per_sub = AGG_ROWS // NS

    @functools.partial(
        pl.kernel, mesh=mesh,
        out_type=jax.ShapeDtypeStruct((NC, AGG_ROWS, HIDDEN), jnp.float32),
        scratch_types=[
            pltpu.VMEM((SFULL, SCB), jnp.int32),
            pltpu.VMEM((STAIL,), jnp.int32),
            pltpu.VMEM((2, SCB, HIDDEN), jnp.float32),
            pltpu.VMEM((STAIL, HIDDEN), jnp.float32),
            pltpu.VMEM_SHARED((AGG_ROWS, HIDDEN), jnp.float32),
            pltpu.SemaphoreType.DMA((2,)),
            pltpu.SemaphoreType.DMA((2,)),
        ],
    )
    def k(m_hbm, ei_hbm, eit_hbm, z_hbm, out_hbm,
          idx_v, idxt_v, rows_v, rowst_v, agg_sh, msem, ssem):
        cid = lax.axis_index("c")
        sid = lax.axis_index("s")
        wid = sid * NC + cid
        base = wid * EPW
        pltpu.sync_copy(ei_hbm.at[wid], idx_v)
        pltpu.sync_copy(eit_hbm.at[wid], idxt_v)
        pltpu.sync_copy(z_hbm.at[pl.ds(sid * rows_per_sub, rows_per_sub)],
                        agg_sh.at[pl.ds(sid * rows_per_sub, rows_per_sub)])
        plsc.subcore_barrier()

        def outer(t, _):
            for b in range(2):
                j = t * 2 + b

                @pl.when(t > 0)
                def _wait_prev_scatter():
                    pltpu.make_async_copy(
                        rows_v.at[b], agg_sh.at[idx_v.at[j]], ssem.at[b]).wait()

                pltpu.make_async_copy(
                    m_hbm.at[pl.ds(base + j * SCB, SCB)], rows_v.at[b],
                    msem.at[b]).start()
            for b in range(2):
                j = t * 2 + b
                pltpu.make_async_copy(
                    m_hbm.at[pl.ds(base + j * SCB, SCB)], rows_v.at[b],
                    msem.at[b]).wait()
                pltpu.make_async_copy(
                    rows_v.at[b], agg_sh.at[idx_v.at[j]],
                    ssem.at[b]).start(add=True)
            return 0

        lax.fori_loop(0, SOUTER, outer, 0)
        for b in range(2):
            pltpu.make_async_copy(
                rows_v.at[b], agg_sh.at[idx_v.at[b]], ssem.at[b]).wait()
        pltpu.sync_copy(m_hbm.at[pl.ds(base + SFULL * SCB, STAIL)], rowst_v)
        pltpu.sync_copy(rowst_v, agg_sh.at[idxt_v], add=True)
        plsc.subcore_barrier()
        pltpu.sync_copy(agg_sh.at[pl.ds(sid * rows_per_sub, rows_per_sub)],
                        out_hbm.at[cid, pl.ds(sid * rows_per_sub, rows_per_sub)])

    return k(m, ei_main, ei_tail, zeros_nh)


def _tc_edge_mlp(x_j, edge_f, W1a, W1b, b1, W2, b2):
    BE = 16000

    def body(xj_ref, f_ref, w1a, w1b, b1r, w2, b2r, out_ref):
        xjb = xj_ref[...].astype(jnp.bfloat16)
        h = jnp.dot(xjb, w1a[...], preferred_element_type=jnp.float32)
        h = h + jnp.dot(f_ref[...], w1b[...], preferred_element_type=jnp.float32)
        h = h + b1r[...]
        h = h * jax.nn.sigmoid(h)
        hb = h.astype(jnp.bfloat16)
        mm = jnp.dot(hb, w2[...], preferred_element_type=jnp.float32) + b2r[...]
        out_ref[...] = mm * jax.nn.sigmoid(mm)

    return pl.pallas_call(
        body,
        grid=(N_EDGES // BE,),
        in_specs=[
            pl.BlockSpec((BE, ATOM_DIM), lambda i: (i, 0)),
            pl.BlockSpec((BE, EDGE_DIM), lambda i: (i, 0)),
            pl.BlockSpec((ATOM_DIM, HIDDEN), lambda i: (0, 0)),
            pl.BlockSpec((EDGE_DIM, HIDDEN), lambda i: (0, 0)),
            pl.BlockSpec((1, HIDDEN), lambda i: (0, 0)),
            pl.BlockSpec((HIDDEN, HIDDEN), lambda i: (0, 0)),
            pl.BlockSpec((1, HIDDEN), lambda i: (0, 0)),
        ],
        out_specs=pl.BlockSpec((BE, HIDDEN), lambda i: (i, 0)),
        out_shape=jax.ShapeDtypeStruct((N_EDGES, HIDDEN), jnp.float32),
    )(x_j, edge_f, W1a, W1b, b1.reshape(1, HIDDEN), W2, b2.reshape(1, HIDDEN))


def _tc_final(x, agg2, Wl, bl):
    BN = 1000

    def body(x_ref, a0_ref, a1_ref, wl, blr, out_ref):
        a = a0_ref[0] + a1_ref[0]
        t = x_ref[...] + jnp.dot(a, wl[...], preferred_element_type=jnp.float32)
        t = t + blr[...]
        out_ref[...] = t * jax.nn.sigmoid(t)

    return pl.pallas_call(
        body,
        grid=(N_NODES // BN,),
        in_specs=[
            pl.BlockSpec((BN, ATOM_DIM), lambda i: (i, 0)),
            pl.BlockSpec((1, BN, HIDDEN), lambda i: (0, i, 0)),
            pl.BlockSpec((1, BN, HIDDEN), lambda i: (1, i, 0)),
            pl.BlockSpec((HIDDEN, ATOM_DIM), lambda i: (0, 0)),
            pl.BlockSpec((1, ATOM_DIM), lambda i: (0, 0)),
        ],
        out_specs=pl.BlockSpec((BN, ATOM_DIM), lambda i: (i, 0)),
        out_shape=jax.ShapeDtypeStruct((N_NODES, ATOM_DIM), jnp.float32),
    )(x, agg2, agg2, Wl, bl.reshape(1, ATOM_DIM))


def kernel(x, edge_i, edge_j, edge_f, W1, b1, W2, b2, Wl, bl):
    ei2 = edge_i.astype(jnp.int32).reshape(NW, EPW)
    ei_main = ei2[:, :SFULL * SCB].reshape(NW, SFULL, SCB)
    ei_tail = ei2[:, SFULL * SCB:]
    ej2 = edge_j.astype(jnp.int32).reshape(NW, EPW)
    ej_main = ej2[:, :GFULL * GCB].reshape(NW, GFULL, GCB)
    ej_tail = ej2[:, GFULL * GCB:]
    x_j = _sc_gather(x, ej_main, ej_tail)
    m = _tc_edge_mlp(x_j, edge_f.astype(jnp.bfloat16),
                     W1[:ATOM_DIM].astype(jnp.bfloat16),
                     W1[ATOM_DIM:].astype(jnp.bfloat16),
                     b1, W2.astype(jnp.bfloat16), b2)
    zeros = jnp.zeros((AGG_ROWS, HIDDEN), jnp.float32)
    agg2 = _sc_scatter_add(m, ei_main, ei_tail, zeros)
    return _tc_final(x, agg2, Wl, bl)
